# baseline (speedup 1.0000x reference)
import jax
import jax.numpy as jnp
from jax.experimental import pallas as pl


def _router_block(x_ref, wt_ref, b_ref, w_out_ref, i_out_ref):
    logits = jax.lax.dot_general(
        x_ref[...], wt_ref[...],
        dimension_numbers=(((1,), (0,)), ((), ())),
        preferred_element_type=jnp.float32,
    ) + b_ref[...]
    m = jnp.max(logits, axis=1, keepdims=True)
    s = jnp.sum(jnp.exp(logits - m), axis=1)
    w_out_ref[0, 0, :] = 1.0 / s
    iota = jax.lax.broadcasted_iota(jnp.int32, logits.shape, 1)
    idx = jnp.min(jnp.where(logits == m, iota, logits.shape[1]), axis=1)
    i_out_ref[0, 0, :] = idx


def kernel(x, W, b):
    tokens, d_model = x.shape
    num_experts = W.shape[0]
    tile = min(1024, tokens)
    grid = tokens // tile
    wt = W.T
    b2 = b.reshape(1, num_experts)
    weights, indices = pl.pallas_call(
        _router_block,
        grid=(grid,),
        in_specs=[
            pl.BlockSpec((tile, d_model), lambda i: (i, 0)),
            pl.BlockSpec((d_model, num_experts), lambda i: (0, 0)),
            pl.BlockSpec((1, num_experts), lambda i: (0, 0)),
        ],
        out_specs=[
            pl.BlockSpec((1, 1, tile), lambda i: (i, 0, 0)),
            pl.BlockSpec((1, 1, tile), lambda i: (i, 0, 0)),
        ],
        out_shape=[
            jax.ShapeDtypeStruct((grid, 1, tile), jnp.float32),
            jax.ShapeDtypeStruct((grid, 1, tile), jnp.int32),
        ],
    )(x, wt, b2)
    return weights.reshape(tokens), indices.reshape(tokens)
